# baseline (speedup 1.0000x reference)
import jax
import jax.numpy as jnp
from jax.experimental import pallas as pl
from jax.experimental.pallas import tpu as pltpu

B, T, N, F, GF = 512, 12, 200, 16, 1
GH, OUT, FCH, NL = 64, 512, 512, 3
EPS = 1e-5

_WX = 0
_WH = 192
_BL = 384
_WOUT = 387
_CHEB = 515
_W0 = 715
_VECS = 745
_BG = 765
_B1 = 766
_WG = 767


def _sig(x):
    return 0.5 * jnp.tanh(0.5 * x) + 0.5


def _bn_cols(h, g, be):
    m = jnp.mean(h, axis=0, keepdims=True)
    v = jnp.mean((h - m) * (h - m), axis=0, keepdims=True)
    return g * (h - m) * jax.lax.rsqrt(v + EPS) + be


def _stgcn_body(x1t_ref, x2_ref, p_ref, o_ref, seq, zx):
    sid = x2_ref[:, F - 1:F].astype(jnp.int32)
    ids = jax.lax.broadcasted_iota(jnp.int32, (B, N), 1)
    onehot = (ids == sid).astype(jnp.float32)
    rows1 = jnp.dot(onehot, p_ref[_CHEB:_CHEB + N, :N],
                    preferred_element_type=jnp.float32)

    wg0 = p_ref[_WG:_WG + 1, :GH]
    wg1 = p_ref[_WG:_WG + 1, GH:2 * GH]
    bg = p_ref[_BG:_BG + 1, :GH]
    for t in range(T):
        xt = x1t_ref[t]
        s0 = jnp.sum(onehot * xt, axis=1, keepdims=True)
        s1 = jnp.sum(rows1 * xt, axis=1, keepdims=True)
        xg = jnp.maximum(s0 * wg0 + s1 * wg1 + bg, 0.0)
        seq[t * B:(t + 1) * B, :] = xg

    h = jnp.zeros((B, GH), jnp.float32)
    for l in range(NL):
        wh = p_ref[_WH + GH * l:_WH + GH * (l + 1), :]
        zx[...] = jnp.dot(seq[...], p_ref[_WX + GH * l:_WX + GH * (l + 1), :],
                          preferred_element_type=jnp.float32
                          ) + p_ref[_BL + l:_BL + l + 1, :]
        h = jnp.zeros((B, GH), jnp.float32)
        c = jnp.zeros((B, GH), jnp.float32)
        for t in range(T):
            z = zx[t * B:(t + 1) * B, :] + jnp.dot(
                h, wh, preferred_element_type=jnp.float32)
            zi = z[:, :GH]
            zf = z[:, GH:2 * GH]
            zg = z[:, 2 * GH:3 * GH]
            zo = z[:, 3 * GH:]
            c = _sig(zf) * c + _sig(zi) * jnp.tanh(zg)
            h = _sig(zo) * jnp.tanh(c)
            seq[t * B:(t + 1) * B, :] = h

    vec = p_ref[_VECS:_VECS + 20, :]
    x2f = x2_ref[:, :F - 1]
    y = p_ref[_B1:_B1 + 1, :1]
    for blk in range(2):
        gFb = jnp.dot(h, p_ref[_WOUT + GH * blk:_WOUT + GH * (blk + 1), :],
                      preferred_element_type=jnp.float32
                      ) + vec[blk:blk + 1, :]
        h2pb = jnp.dot(x2f, p_ref[_W0 + 15 * blk:_W0 + 15 * (blk + 1), :],
                       preferred_element_type=jnp.float32
                       ) + vec[2 + blk:3 + blk, :]
        h2nb = _bn_cols(h2pb, vec[4 + blk:5 + blk, :], vec[6 + blk:7 + blk, :])
        h2b = jnp.where(h2nb >= 0, h2nb, 0.01 * h2nb)
        bna = _bn_cols(h2b, vec[8 + blk:9 + blk, :], vec[12 + blk:13 + blk, :])
        bnb = _bn_cols(gFb, vec[10 + blk:11 + blk, :], vec[14 + blk:15 + blk, :])
        y = y + (jnp.sum(bna * vec[16 + blk:17 + blk, :], axis=1, keepdims=True)
                 + jnp.sum(bnb * vec[18 + blk:19 + blk, :], axis=1, keepdims=True))
    o_ref[...] = _sig(y)


_CALL = pl.pallas_call(
    _stgcn_body,
    out_shape=jax.ShapeDtypeStruct((B, 1), jnp.float32),
    scratch_shapes=[
        pltpu.VMEM((T * B, GH), jnp.float32),
        pltpu.VMEM((T * B, 4 * GH), jnp.float32),
    ],
    compiler_params=pltpu.CompilerParams(
        vmem_limit_bytes=100 * 1024 * 1024,
    ),
)


def kernel(x, cheb, W_g, b_g, Wx, Wh, b_lstm, W_out, b_out, W0, b0, g0, be0,
           g1, be1, W1, b1):
    x1t = x[:, :T * N * GF].reshape(B, T, N).transpose(1, 0, 2)
    x2 = x[:, T * N * GF:]
    w1f = W1.reshape(FCH + OUT)
    params = jnp.concatenate([
        Wx.reshape(NL * GH, 4 * GH),
        Wh.reshape(NL * GH, 4 * GH),
        b_lstm,
        W_out[:, :256], W_out[:, 256:],
        jnp.pad(cheb[1], ((0, 0), (0, 256 - N))),
        W0[:, :256], W0[:, 256:],
        b_out.reshape(2, 256), b0.reshape(2, 256),
        g0.reshape(2, 256), be0.reshape(2, 256),
        g1.reshape(4, 256), be1.reshape(4, 256), w1f.reshape(4, 256),
        jnp.pad(b_g.reshape(1, GH), ((0, 0), (0, 256 - GH))),
        jnp.pad(b1.reshape(1, 1), ((0, 0), (0, 255))),
        jnp.pad(W_g.reshape(1, 2 * GH), ((0, 0), (0, 128))),
    ], axis=0)
    return _CALL(x1t, x2, params)


# speedup vs baseline: 1.9107x; 1.9107x over previous
import jax
import jax.numpy as jnp
from jax.experimental import pallas as pl
from jax.experimental.pallas import tpu as pltpu

B, T, N, F, GF = 512, 12, 200, 16, 1
GH, OUT, FCH, NL = 64, 512, 512, 3
EPS = 1e-5
HB = B // 2


def _bn_cols(h, g, be):
    m = jnp.mean(h, axis=0, keepdims=True)
    v = jnp.mean((h - m) * (h - m), axis=0, keepdims=True)
    return g * (h - m) * jax.lax.rsqrt(v + EPS) + be


def _stgcn_body(x1t_ref, x2_ref, cheb1_ref, wg_ref, bg_ref,
                wx_ref, wh_ref, bl_ref, wout_ref, bout_ref,
                w0_ref, b0_ref, g0_ref, be0_ref,
                g1_ref, be1_ref, w1_ref, b1_ref,
                o_ref, seq, zx):
    sid = x2_ref[:, F - 1:F].astype(jnp.int32)
    ids = jax.lax.broadcasted_iota(jnp.int32, (B, N), 1)
    onehot = (ids == sid).astype(jnp.float32)
    rows1 = jnp.dot(onehot, cheb1_ref[...],
                    preferred_element_type=jnp.float32)

    wg0 = wg_ref[0]
    wg1 = wg_ref[1]
    bg = bg_ref[...]
    for t in range(T):
        xt = x1t_ref[t]
        s0 = jnp.sum(onehot * xt, axis=1, keepdims=True)
        s1 = jnp.sum(rows1 * xt, axis=1, keepdims=True)
        xg = jnp.maximum(s0 * wg0 + s1 * wg1 + bg, 0.0)
        seq[t * B:(t + 1) * B, :] = xg

    lane = jax.lax.broadcasted_iota(jnp.int32, (1, 4 * GH), 1)
    gs = jnp.where(lane < 3 * GH, 0.5, 1.0)

    def _step(zslice, h, c, wh_s):
        z = zslice + jnp.dot(h, wh_s, preferred_element_type=jnp.float32)
        ti = jnp.tanh(z[:, :GH])
        tf = jnp.tanh(z[:, GH:2 * GH])
        tg = jnp.tanh(z[:, 2 * GH:3 * GH])
        to = jnp.tanh(z[:, 3 * GH:])
        c = 0.5 * (tf * c + c + ti * tg + tg)
        tc = jnp.tanh(c)
        return to * tc + tc, c

    hA = jnp.zeros((HB, GH), jnp.float32)
    hB = jnp.zeros((HB, GH), jnp.float32)
    for l in range(NL):
        wx_s = wx_ref[l] * (gs if l == 0 else 0.5 * gs)
        wh_s = wh_ref[l] * (0.5 * gs)
        zx[...] = jnp.dot(seq[...], wx_s,
                          preferred_element_type=jnp.float32
                          ) + bl_ref[l:l + 1, :] * gs
        hA = jnp.zeros((HB, GH), jnp.float32)
        cA = jnp.zeros((HB, GH), jnp.float32)
        hB = jnp.zeros((HB, GH), jnp.float32)
        cB = jnp.zeros((HB, GH), jnp.float32)
        for t in range(T):
            base = t * B
            hA, cA = _step(zx[base:base + HB, :], hA, cA, wh_s)
            hB, cB = _step(zx[base + HB:base + B, :], hB, cB, wh_s)
            seq[base:base + HB, :] = hA
            seq[base + HB:base + B, :] = hB

    wout_s = wout_ref[...] * 0.5
    gF = jnp.concatenate([
        jnp.dot(hA, wout_s, preferred_element_type=jnp.float32),
        jnp.dot(hB, wout_s, preferred_element_type=jnp.float32),
    ], axis=0) + bout_ref[...]

    h2p = jnp.dot(x2_ref[:, :F - 1], w0_ref[...],
                  preferred_element_type=jnp.float32) + b0_ref[...]
    h2n = _bn_cols(h2p, g0_ref[...], be0_ref[...])
    h2 = jnp.where(h2n >= 0, h2n, 0.01 * h2n)
    bna = _bn_cols(h2, g1_ref[:, :FCH], be1_ref[:, :FCH])
    bnb = _bn_cols(gF, g1_ref[:, FCH:], be1_ref[:, FCH:])
    y = (jnp.sum(bna * w1_ref[:, :FCH], axis=1, keepdims=True)
         + jnp.sum(bnb * w1_ref[:, FCH:], axis=1, keepdims=True)
         + b1_ref[...])
    o_ref[...] = 0.5 * jnp.tanh(0.5 * y) + 0.5


_CALL = pl.pallas_call(
    _stgcn_body,
    out_shape=jax.ShapeDtypeStruct((B, 1), jnp.float32),
    scratch_shapes=[
        pltpu.VMEM((T * B, GH), jnp.float32),
        pltpu.VMEM((T * B, 4 * GH), jnp.float32),
    ],
    compiler_params=pltpu.CompilerParams(
        vmem_limit_bytes=100 * 1024 * 1024,
    ),
)


def kernel(x, cheb, W_g, b_g, Wx, Wh, b_lstm, W_out, b_out, W0, b0, g0, be0,
           g1, be1, W1, b1):
    x1t = x[:, :T * N * GF].reshape(B, T, N).transpose(1, 0, 2)
    x2 = x[:, T * N * GF:]
    return _CALL(x1t, x2, cheb[1], W_g, b_g.reshape(1, GH), Wx, Wh, b_lstm,
                 W_out, b_out.reshape(1, OUT),
                 W0, b0.reshape(1, FCH), g0.reshape(1, FCH),
                 be0.reshape(1, FCH),
                 g1.reshape(1, FCH + OUT), be1.reshape(1, FCH + OUT),
                 W1.reshape(1, FCH + OUT), b1.reshape(1, 1))


# speedup vs baseline: 1.9144x; 1.0019x over previous
import jax
import jax.numpy as jnp
from jax.experimental import pallas as pl
from jax.experimental.pallas import tpu as pltpu

B, T, N, F, GF = 512, 12, 200, 16, 1
GH, OUT, FCH, NL = 64, 512, 512, 3
EPS = 1e-5
HB = B // 2


def _bn_cols(h, g, be):
    m = jnp.mean(h, axis=0, keepdims=True)
    v = jnp.mean((h - m) * (h - m), axis=0, keepdims=True)
    return g * (h - m) * jax.lax.rsqrt(v + EPS) + be


def _stgcn_body(x1t_ref, x2_ref, cheb1_ref, wg_ref, bg_ref,
                wx_ref, wh_ref, bl_ref, wout_ref, bout_ref,
                w0_ref, b0_ref, g0_ref, be0_ref,
                g1_ref, be1_ref, w1_ref, b1_ref,
                o_ref, seq, zx):
    sid = x2_ref[:, F - 1:F].astype(jnp.int32)
    ids = jax.lax.broadcasted_iota(jnp.int32, (B, N), 1)
    onehot = (ids == sid).astype(jnp.float32)
    rows1 = jnp.dot(onehot, cheb1_ref[...],
                    preferred_element_type=jnp.float32)

    wg0 = wg_ref[0]
    wg1 = wg_ref[1]
    bg = bg_ref[...]
    for t in range(T):
        xt = x1t_ref[t]
        s0 = jnp.sum(onehot * xt, axis=1, keepdims=True)
        s1 = jnp.sum(rows1 * xt, axis=1, keepdims=True)
        xg = jnp.maximum(s0 * wg0 + s1 * wg1 + bg, 0.0)
        seq[t * B:(t + 1) * B, :] = xg

    lane = jax.lax.broadcasted_iota(jnp.int32, (1, 4 * GH), 1)
    gs = jnp.where((lane < 2 * GH) | (lane >= 3 * GH), 0.5, 1.0)

    def _step(zslice, h, c, wh_s):
        z = zslice + jnp.dot(h, wh_s, preferred_element_type=jnp.float32)
        ti = jnp.tanh(z[:, :GH])
        tf = jnp.tanh(z[:, GH:2 * GH])
        tg = jnp.tanh(z[:, 2 * GH:3 * GH])
        to = jnp.tanh(z[:, 3 * GH:])
        c = 0.5 * (tf * c + c + ti * tg + tg)
        tc = jnp.tanh(c)
        return to * tc + tc, c

    hA = jnp.zeros((HB, GH), jnp.float32)
    hB = jnp.zeros((HB, GH), jnp.float32)
    for l in range(NL):
        wx_s = wx_ref[l] * (gs if l == 0 else 0.5 * gs)
        wh_s = wh_ref[l] * (0.5 * gs)
        zx[...] = jnp.dot(seq[...], wx_s,
                          preferred_element_type=jnp.float32
                          ) + bl_ref[l:l + 1, :] * gs
        hA = jnp.zeros((HB, GH), jnp.float32)
        cA = jnp.zeros((HB, GH), jnp.float32)
        hB = jnp.zeros((HB, GH), jnp.float32)
        cB = jnp.zeros((HB, GH), jnp.float32)
        for t in range(T):
            base = t * B
            hA, cA = _step(zx[base:base + HB, :], hA, cA, wh_s)
            hB, cB = _step(zx[base + HB:base + B, :], hB, cB, wh_s)
            seq[base:base + HB, :] = hA
            seq[base + HB:base + B, :] = hB

    wout_s = wout_ref[...] * 0.5
    gF = jnp.concatenate([
        jnp.dot(hA, wout_s, preferred_element_type=jnp.float32),
        jnp.dot(hB, wout_s, preferred_element_type=jnp.float32),
    ], axis=0) + bout_ref[...]

    h2p = jnp.dot(x2_ref[:, :F - 1], w0_ref[...],
                  preferred_element_type=jnp.float32) + b0_ref[...]
    h2n = _bn_cols(h2p, g0_ref[...], be0_ref[...])
    h2 = jnp.where(h2n >= 0, h2n, 0.01 * h2n)
    bna = _bn_cols(h2, g1_ref[:, :FCH], be1_ref[:, :FCH])
    bnb = _bn_cols(gF, g1_ref[:, FCH:], be1_ref[:, FCH:])
    y = (jnp.sum(bna * w1_ref[:, :FCH], axis=1, keepdims=True)
         + jnp.sum(bnb * w1_ref[:, FCH:], axis=1, keepdims=True)
         + b1_ref[...])
    o_ref[...] = 0.5 * jnp.tanh(0.5 * y) + 0.5


_CALL = pl.pallas_call(
    _stgcn_body,
    out_shape=jax.ShapeDtypeStruct((B, 1), jnp.float32),
    scratch_shapes=[
        pltpu.VMEM((T * B, GH), jnp.float32),
        pltpu.VMEM((T * B, 4 * GH), jnp.float32),
    ],
    compiler_params=pltpu.CompilerParams(
        vmem_limit_bytes=100 * 1024 * 1024,
    ),
)


def kernel(x, cheb, W_g, b_g, Wx, Wh, b_lstm, W_out, b_out, W0, b0, g0, be0,
           g1, be1, W1, b1):
    x1t = x[:, :T * N * GF].reshape(B, T, N).transpose(1, 0, 2)
    x2 = x[:, T * N * GF:]
    return _CALL(x1t, x2, cheb[1], W_g, b_g.reshape(1, GH), Wx, Wh, b_lstm,
                 W_out, b_out.reshape(1, OUT),
                 W0, b0.reshape(1, FCH), g0.reshape(1, FCH),
                 be0.reshape(1, FCH),
                 g1.reshape(1, FCH + OUT), be1.reshape(1, FCH + OUT),
                 W1.reshape(1, FCH + OUT), b1.reshape(1, 1))
